# baseline (speedup 1.0000x reference)
import functools

import jax
import jax.numpy as jnp
from jax import lax
from jax.experimental import pallas as pl
from jax.experimental.pallas import tpu as pltpu
from jax.experimental.pallas import tpu_sc as plsc

_N = 2_000_000
_S = 50_000
_SP = 50_048
_C = 3200
_NW = 32
_TOTAL_CHUNKS = _N // _C
_MAX_SLOTS = -(-_TOTAL_CHUNKS // _NW)
_L = 16
_MASK_HI = -65536
_SL = _SP // 16
_SL_LAST = _S - 15 * _SL
_SLP = 3136
_PREP_ITERS = _SLP // _L


def _packv(hi, lo):
    hb = lax.bitcast_convert_type(hi, jnp.int32)
    lb = lax.bitcast_convert_type(lo, jnp.int32)
    packed = ((hb + 0x8000) & _MASK_HI) | lax.shift_right_logical(lb + 0x8000, 16)
    return lax.bitcast_convert_type(packed, jnp.float32)


def _make_sc_kernel():
    mesh = plsc.VectorSubcoreMesh(core_axis_name="c", subcore_axis_name="s")

    @functools.partial(
        pl.kernel,
        mesh=mesh,
        compiler_params=pltpu.CompilerParams(needs_layout_passes=False),
        out_type=(
            jax.ShapeDtypeStruct((_NW, _L), jnp.float32),
            jax.ShapeDtypeStruct((2 * _SP,), jnp.float32),
            jax.ShapeDtypeStruct((2 * _SP,), jnp.float32),
        ),
        scratch_types=[
            pltpu.VMEM((_SP,), jnp.float32),
            pltpu.VMEM((_SP,), jnp.float32),
            pltpu.VMEM((_C,), jnp.float32),
            pltpu.VMEM((_C,), jnp.float32),
            pltpu.VMEM((_C,), jnp.float32),
            pltpu.VMEM((_C,), jnp.int32),
            pltpu.VMEM((_C,), jnp.float32),
            pltpu.VMEM((_C,), jnp.float32),
            pltpu.VMEM((_C,), jnp.float32),
            pltpu.VMEM((_C,), jnp.int32),
            pltpu.VMEM((_L,), jnp.float32),
            pltpu.SemaphoreType.DMA,
            pltpu.SemaphoreType.DMA,
            pltpu.SemaphoreType.DMA,
        ],
    )
    def sc_partial(y_hbm, j_hbm, k_hbm, sub_hbm, a_hbm, u_hbm, lam_hbm, gam_hbm,
                   out_hbm, t1_hbm, t2_hbm, p1_v, p2_v,
                   ya, ja, ka, sa, yb, jb, kb, sb, acc_v, sema, semb, semt):
        cid = lax.axis_index("c")
        sid = lax.axis_index("s")
        wid = sid * 2 + cid

        def slot_base(s):
            return pl.multiple_of((wid + s * _NW) * _C, 8)

        def issue(s, yv, jv, kv, sv, sem):
            @pl.when(wid + s * _NW < _TOTAL_CHUNKS)
            def _():
                base = slot_base(s)
                pltpu.async_copy(y_hbm.at[pl.ds(base, _C)], yv, sem)
                pltpu.async_copy(j_hbm.at[pl.ds(base, _C)], jv, sem)
                pltpu.async_copy(k_hbm.at[pl.ds(base, _C)], kv, sem)
                pltpu.async_copy(sub_hbm.at[pl.ds(base, _C)], sv, sem)

        acc_v[...] = jnp.zeros((_L,), jnp.float32)
        issue(0, ya, ja, ka, sa, sema)
        issue(1, yb, jb, kb, sb, semb)

        off = sid * _SL
        soff = pl.multiple_of(cid * _SP + off, 8)

        def stage_in(ln):
            pltpu.async_copy(a_hbm.at[pl.ds(off, ln)], p1_v.at[pl.ds(0, ln)], semt)
            pltpu.async_copy(u_hbm.at[pl.ds(off, ln)], p1_v.at[pl.ds(_SLP, ln)], semt)
            pltpu.async_copy(lam_hbm.at[pl.ds(off, ln)], p1_v.at[pl.ds(2 * _SLP, ln)], semt)
            pltpu.async_copy(gam_hbm.at[pl.ds(off, ln)], p1_v.at[pl.ds(3 * _SLP, ln)], semt)

        def stage_wait(ln):
            pltpu.make_async_copy(a_hbm.at[pl.ds(off, ln)], p1_v.at[pl.ds(0, ln)], semt).wait()
            pltpu.make_async_copy(u_hbm.at[pl.ds(off, ln)], p1_v.at[pl.ds(_SLP, ln)], semt).wait()
            pltpu.make_async_copy(lam_hbm.at[pl.ds(off, ln)], p1_v.at[pl.ds(2 * _SLP, ln)], semt).wait()
            pltpu.make_async_copy(gam_hbm.at[pl.ds(off, ln)], p1_v.at[pl.ds(3 * _SLP, ln)], semt).wait()

        def stage_out(ln):
            pltpu.async_copy(p2_v.at[pl.ds(0, ln)], t1_hbm.at[pl.ds(soff, ln)], semt)
            pltpu.async_copy(p2_v.at[pl.ds(_SLP, ln)], t2_hbm.at[pl.ds(soff, ln)], semt)
            pltpu.make_async_copy(p2_v.at[pl.ds(0, ln)], t1_hbm.at[pl.ds(soff, ln)], semt).wait()
            pltpu.make_async_copy(p2_v.at[pl.ds(_SLP, ln)], t2_hbm.at[pl.ds(soff, ln)], semt).wait()

        last = sid == 15

        @pl.when(jnp.logical_not(last))
        def _():
            stage_in(_SL)
            stage_wait(_SL)

        @pl.when(last)
        def _():
            stage_in(_SL_LAST)
            stage_wait(_SL_LAST)

        @plsc.parallel_loop(0, _PREP_ITERS, unroll=4)
        def _(t):
            sl = pl.ds(t * _L, _L)
            a_ = jnp.maximum(p1_v[sl], 0.0)
            u_ = jnp.maximum(p1_v[pl.ds(_SLP + t * _L, _L)], 0.0)
            el = jnp.exp(-p1_v[pl.ds(2 * _SLP + t * _L, _L)])
            eg = jnp.exp(-p1_v[pl.ds(3 * _SLP + t * _L, _L)])
            l_ = 0.2 / (1.0 + el)
            lg = l_ / (1.0 + eg)
            p2_v[sl] = _packv(a_, u_)
            p2_v[pl.ds(_SLP + t * _L, _L)] = _packv(l_, lg)

        @pl.when(jnp.logical_not(last))
        def _():
            stage_out(_SL)

        @pl.when(last)
        def _():
            stage_out(_SL_LAST)

        plsc.subcore_barrier()

        tbase = pl.multiple_of(cid * _SP, 8)
        pltpu.async_copy(t1_hbm.at[pl.ds(tbase, _S)], p1_v.at[pl.ds(0, _S)], semt)
        pltpu.async_copy(t2_hbm.at[pl.ds(tbase, _S)], p2_v.at[pl.ds(0, _S)], semt)
        pltpu.make_async_copy(t1_hbm.at[pl.ds(tbase, _S)], p1_v.at[pl.ds(0, _S)], semt).wait()
        pltpu.make_async_copy(t2_hbm.at[pl.ds(tbase, _S)], p2_v.at[pl.ds(0, _S)], semt).wait()

        def consume(s, yv, jv, kv, sv, sem):
            @pl.when(wid + s * _NW < _TOTAL_CHUNKS)
            def _():
                base = slot_base(s)
                pltpu.make_async_copy(y_hbm.at[pl.ds(base, _C)], yv, sem).wait()
                pltpu.make_async_copy(j_hbm.at[pl.ds(base, _C)], jv, sem).wait()
                pltpu.make_async_copy(k_hbm.at[pl.ds(base, _C)], kv, sem).wait()
                pltpu.make_async_copy(sub_hbm.at[pl.ds(base, _C)], sv, sem).wait()

                def vec_body(t, a2):
                    sl = pl.ds(t * _L, _L)
                    idx = sv[sl]
                    w1 = lax.bitcast_convert_type(
                        plsc.load_gather(p1_v, [idx]), jnp.int32)
                    w2 = lax.bitcast_convert_type(
                        plsc.load_gather(p2_v, [idx]), jnp.int32)
                    a_ = lax.bitcast_convert_type(w1 & _MASK_HI, jnp.float32)
                    u_ = lax.bitcast_convert_type(w1 << 16, jnp.float32)
                    l_ = lax.bitcast_convert_type(w2 & _MASK_HI, jnp.float32)
                    lg = lax.bitcast_convert_type(w2 << 16, jnp.float32)
                    mu = a_ - u_ * jnp.exp(-(l_ * jv[sl] + lg * kv[sl]))
                    d = yv[sl] - mu
                    return a2 + d * d

                contrib = plsc.parallel_loop(
                    0, _C // _L, unroll=8,
                    carry=jnp.zeros((_L,), jnp.float32))(vec_body)
                acc_v[...] = acc_v[...] + contrib

        def pair_body(p, carry):
            s0 = 2 * p
            consume(s0, ya, ja, ka, sa, sema)
            issue(s0 + 2, ya, ja, ka, sa, sema)
            consume(s0 + 1, yb, jb, kb, sb, semb)
            issue(s0 + 3, yb, jb, kb, sb, semb)
            return carry

        lax.fori_loop(0, _MAX_SLOTS // 2, pair_body, jnp.int32(0))
        pltpu.sync_copy(acc_v, out_hbm.at[wid])

    return sc_partial


_sc_partial = _make_sc_kernel()


def _finish_body(p_ref, o_ref):
    o_ref[0, 0] = jnp.sqrt(jnp.sum(p_ref[...]) / _N)


_finish = pl.pallas_call(
    _finish_body,
    out_shape=jax.ShapeDtypeStruct((1, 1), jnp.float32),
    out_specs=pl.BlockSpec(memory_space=pltpu.SMEM),
)


def kernel(y, j, k, sub, A, U, Lambda, Gamma1):
    partials, _, _ = _sc_partial(y, j, k, sub.astype(jnp.int32),
                                 A, U, Lambda, Gamma1)
    return _finish(partials)[0, 0]
